# baseline (speedup 1.0000x reference)
import functools

import jax
import jax.numpy as jnp
from jax import lax
from jax.experimental import pallas as pl
from jax.experimental.pallas import tpu as pltpu
from jax.experimental.pallas import tpu_sc as plsc

NC = 2
NS = 16
LANES = 16
NW = NC * NS

BATCH = 4096
SEQ = 200
EMBED = 64
CH0 = 128
CH1 = SEQ - CH0


def _pooled_sc(x_flat, table):
    b_per_w = BATCH // NW
    mesh = plsc.VectorSubcoreMesh(core_axis_name="c", subcore_axis_name="s")

    @functools.partial(
        pl.kernel,
        out_type=jax.ShapeDtypeStruct((BATCH, EMBED), jnp.float32),
        mesh=mesh,
        scratch_types=[
            pltpu.VMEM((b_per_w * SEQ,), jnp.int32),
            pltpu.VMEM((CH0, EMBED), jnp.float32),
            pltpu.VMEM((b_per_w, EMBED), jnp.float32),
            pltpu.SemaphoreType.DMA,
        ],
        compiler_params=pltpu.CompilerParams(use_tc_tiling_on_sc=False),
    )
    def k(x_hbm, table_hbm, out_hbm, idx_v, rows_v, pooled_v, sem):
        wid = lax.axis_index("s") * NC + lax.axis_index("c")
        row0 = wid * b_per_w
        pltpu.sync_copy(x_hbm.at[pl.ds(row0 * SEQ, b_per_w * SEQ)], idx_v)

        inv = jnp.full((LANES,), 1.0 / SEQ, jnp.float32)

        def acc_chunk(n, a):
            def body(j, a):
                return tuple(
                    a[g] + rows_v[j, pl.ds(g * LANES, LANES)]
                    for g in range(EMBED // LANES)
                )
            return lax.fori_loop(0, n, body, a)

        def row_body(r, carry):
            off = pl.multiple_of(r * SEQ, 8)
            pltpu.async_copy(
                table_hbm.at[idx_v.at[pl.ds(off, CH0)]],
                rows_v, sem).wait()
            zero = jnp.zeros((LANES,), jnp.float32)
            a = acc_chunk(CH0, (zero,) * (EMBED // LANES))
            off1 = pl.multiple_of(r * SEQ + CH0, 8)
            pltpu.async_copy(
                table_hbm.at[idx_v.at[pl.ds(off1, CH1)]],
                rows_v.at[pl.ds(0, CH1)], sem).wait()
            a = acc_chunk(CH1, a)
            for g in range(EMBED // LANES):
                pooled_v[r, pl.ds(g * LANES, LANES)] = a[g] * inv
            return carry

        lax.fori_loop(0, b_per_w, row_body, 0)
        pltpu.sync_copy(pooled_v, out_hbm.at[pl.ds(row0, b_per_w)])

    return k(x_flat, table)


def _mlp_tc(pooled, W1, b1r, W2r, b2r):
    def body(p_ref, w1_ref, b1_ref, w2_ref, b2_ref, o_ref):
        h = jnp.dot(p_ref[:], w1_ref[:], preferred_element_type=jnp.float32)
        h = jnp.maximum(h + b1_ref[:], 0.0)
        o_ref[:] = jnp.sum(h * w2_ref[:], axis=1) + b2_ref[0, 0]

    return pl.pallas_call(
        body,
        out_shape=jax.ShapeDtypeStruct((BATCH,), jnp.float32),
        in_specs=[
            pl.BlockSpec(memory_space=pltpu.VMEM),
            pl.BlockSpec(memory_space=pltpu.VMEM),
            pl.BlockSpec(memory_space=pltpu.VMEM),
            pl.BlockSpec(memory_space=pltpu.VMEM),
            pl.BlockSpec(memory_space=pltpu.SMEM),
        ],
        out_specs=pl.BlockSpec(memory_space=pltpu.VMEM),
    )(pooled, W1, b1r, W2r, b2r)


def kernel(x, table, W1, b1, W2, b2):
    x_flat = x.reshape(-1).astype(jnp.int32)
    pooled = _pooled_sc(x_flat, table)
    b1r = b1.reshape(1, -1)
    W2r = W2.reshape(1, -1)
    b2r = b2.reshape(1, 1)
    return _mlp_tc(pooled, W1, b1r, W2r, b2r)
